# baseline (speedup 1.0000x reference)
import functools

import jax
import jax.numpy as jnp
from jax import lax
from jax.experimental import pallas as pl
from jax.experimental.pallas import tpu as pltpu
from jax.experimental.pallas import tpu_sc as plsc

L = 200
B = 1024
V = 100000
NCAT = 8
D = 32
NNUM = 4
NED = 32
ROWS = L * B
NG = NNUM + 1
CY = NG * NED
CTOT = NCAT * D + CY
CT = CTOT // 8
LT = L // 8
BT = B // 128

_info = plsc.get_sparse_core_info()
NC, NS = _info.num_cores, _info.num_subcores
NW = NC * NS
NCHUNK = (L * BT) // NW


def _sc_body(c0, c1, c2, c3, c4, c5, c6, c7,
             xt, wrep, brep,
             t0, t1, t2, t3, t4, t5, t6, t7,
             out,
             ia0, ia1, ia2, ia3, ia4, ia5, ia6, ia7,
             ib0, ib1, ib2, ib3, ib4, ib5, ib6, ib7,
             g0, g1, g2, g3, g4, g5, g6, g7,
             asm, xba, xbb, wbuf, bbuf,
             isa, isb, gsem, wsem):
    cats = [c0, c1, c2, c3, c4, c5, c6, c7]
    tables = [t0, t1, t2, t3, t4, t5, t6, t7]
    gbufs = [g0, g1, g2, g3, g4, g5, g6, g7]
    sets = [
        dict(idx=[ia0, ia1, ia2, ia3, ia4, ia5, ia6, ia7],
             xbuf=xba, isem=isa),
        dict(idx=[ib0, ib1, ib2, ib3, ib4, ib5, ib6, ib7],
             xbuf=xbb, isem=isb),
    ]

    wid = lax.axis_index("s") * NC + lax.axis_index("c")

    pltpu.sync_copy(wrep, wbuf)
    pltpu.sync_copy(brep, bbuf)

    iota = lax.iota(jnp.int32, 16)

    def coords(j):
        cid = wid * NCHUNK + j
        l = cid // BT
        bt = cid - l * BT
        return l, bt

    def idx_copies(j, s):
        l, bt = coords(j)
        lt = l // 8
        sub = l - lt * 8
        cps = [pltpu.make_async_copy(cats[i].at[lt, bt, sub, :],
                                     s["idx"][i], s["isem"])
               for i in range(NCAT)]
        cps.append(pltpu.make_async_copy(
            xt.at[:, pl.ds(l * B + bt * 128, 128)], s["xbuf"], s["isem"]))
        return cps

    def start(j, s):
        for cp in idx_copies(j, s):
            cp.start()

    def out_copy(j):
        l, bt = coords(j)
        return pltpu.make_async_copy(asm, out.at[l, :, pl.ds(bt, 1), :, :],
                                     wsem)

    def finish(j, s):
        with jax.named_scope("idxwait"):
            for cp in idx_copies(j, s):
                cp.wait()
            for i in range(NCAT):
                pltpu.async_copy(tables[i].at[s["idx"][i]], gbufs[i], gsem)

        with jax.named_scope("outdrain"):
            @pl.when(j >= 1)
            def _():
                out_copy(j).wait()

        with jax.named_scope("numeric"):
            for g in range(NG):
                xg = [s["xbuf"][g, pl.ds(16 * m, 16)] for m in range(8)]

                def num_body(k, _c, g=g, xg=xg):
                    cn = g * NED + k
                    wvec = wbuf[cn, :]
                    bvec = bbuf[cn, :]
                    ctg = 4 * NCAT + (cn >> 3)
                    cs = cn & 7
                    for m in range(8):
                        asm[ctg, 0, cs, pl.ds(16 * m, 16)] = (
                            xg[m] * wvec + bvec)
                    return _c
                lax.fori_loop(0, NED, num_body, 0)

        with jax.named_scope("transpose"):
            for w in range(NCAT):
                gb = gbufs[w]
                with jax.named_scope("gdrain"):
                    pltpu.make_async_copy(tables[w].at[s["idx"][w]], gb,
                                          gsem).wait()

                def tr_body(m, _c, gb=gb, w=w):
                    rowv = 16 * m + iota
                    for c in range(D):
                        colv = jnp.full((16,), c, jnp.int32)
                        v = plsc.load_gather(gb, [rowv, colv])
                        asm[4 * w + (c >> 3), 0, c & 7, pl.ds(16 * m, 16)] = v
                    return _c
                lax.fori_loop(0, 8, tr_body, 0)

        out_copy(j).start()

    start(0, sets[0])
    start(1, sets[1])

    def iter_body(k, _c):
        finish(2 * k, sets[0])

        @pl.when(k < NCHUNK // 2 - 1)
        def _():
            start(2 * k + 2, sets[0])
        finish(2 * k + 1, sets[1])

        @pl.when(k < NCHUNK // 2 - 1)
        def _():
            start(2 * k + 3, sets[1])
        return _c

    lax.fori_loop(0, NCHUNK // 2, iter_body, 0)

    out_copy(NCHUNK - 1).wait()


@jax.jit
def _sc_call(cats_t, xt, tables, wrep, brep):
    mesh = plsc.VectorSubcoreMesh(core_axis_name="c", subcore_axis_name="s")
    scratch = (
        [pltpu.VMEM((128,), jnp.int32) for _ in range(2 * NCAT)]
        + [pltpu.VMEM((128, D), jnp.float32) for _ in range(NCAT)]
        + [pltpu.VMEM((CT, 1, 8, 128), jnp.float32)]
        + [pltpu.VMEM((NG, 128), jnp.float32) for _ in range(2)]
        + [pltpu.VMEM((CY, 16), jnp.float32),
           pltpu.VMEM((CY, 16), jnp.float32)]
        + [pltpu.SemaphoreType.DMA for _ in range(4)]
    )
    fn = pl.kernel(
        _sc_body,
        out_type=jax.ShapeDtypeStruct((L, CT, BT, 8, 128), jnp.float32),
        mesh=mesh,
        scratch_types=scratch,
        compiler_params=pltpu.CompilerParams(use_tc_tiling_on_sc=False,
                                             needs_layout_passes=False),
    )
    return fn(*cats_t, xt, wrep, brep, *tables)


def kernel(cat0, cat1, cat2, cat3, cat4, cat5, cat6, cat7,
           num_features, time, lengths,
           table0, table1, table2, table3, table4, table5, table6, table7,
           W, b):
    cats_t = [c.astype(jnp.int32).reshape(LT, 8, BT, 128).swapaxes(1, 2)
              for c in (cat0, cat1, cat2, cat3, cat4, cat5, cat6, cat7)]
    tables = [table0, table1, table2, table3, table4, table5, table6, table7]
    xt = jnp.concatenate(
        [num_features.reshape(ROWS, NNUM).T,
         time.reshape(1, ROWS).astype(jnp.float32)], axis=0)
    wrep = jnp.broadcast_to(W.reshape(CY, 1), (CY, 16))
    brep = jnp.broadcast_to(b.reshape(CY, 1), (CY, 16))
    out5 = _sc_call(cats_t, xt, tables, wrep, brep)
    return out5.transpose(0, 2, 4, 1, 3).reshape(L, B, CTOT)


# speedup vs baseline: 1.3556x; 1.3556x over previous
import functools

import jax
import jax.numpy as jnp
from jax import lax
from jax.experimental import pallas as pl
from jax.experimental.pallas import tpu as pltpu
from jax.experimental.pallas import tpu_sc as plsc

L = 200
B = 1024
V = 100000
NCAT = 8
D = 32
NNUM = 4
NED = 32
ROWS = L * B
NG = NNUM + 1
CY = NG * NED
CTOT = NCAT * D + CY
LT = L // 8
BT = B // 128

_info = plsc.get_sparse_core_info()
NC, NS = _info.num_cores, _info.num_subcores
NW = NC * NS
RPW = ROWS // NW
CHUNK = 128
NCHUNK = RPW // CHUNK


def _sc_body(c0, c1, c2, c3, c4, c5, c6, c7,
             xpad, wflat, bflat,
             t0, t1, t2, t3, t4, t5, t6, t7,
             out,
             ia0, ia1, ia2, ia3, ia4, ia5, ia6, ia7,
             ib0, ib1, ib2, ib3, ib4, ib5, ib6, ib7,
             ga0, ga1, ga2, ga3, ga4, ga5, ga6, ga7,
             gb0, gb1, gb2, gb3, gb4, gb5, gb6, gb7,
             xa, xb, ya, yb, wbuf, bbuf,
             isa, isb, gsema, gsemb, wsema, wsemb):
    cats = [c0, c1, c2, c3, c4, c5, c6, c7]
    tables = [t0, t1, t2, t3, t4, t5, t6, t7]
    sets = [
        dict(idx=[ia0, ia1, ia2, ia3, ia4, ia5, ia6, ia7],
             gbuf=[ga0, ga1, ga2, ga3, ga4, ga5, ga6, ga7],
             xbuf=xa, ybuf=ya, isem=isa, gsem=gsema, wsem=wsema),
        dict(idx=[ib0, ib1, ib2, ib3, ib4, ib5, ib6, ib7],
             gbuf=[gb0, gb1, gb2, gb3, gb4, gb5, gb6, gb7],
             xbuf=xb, ybuf=yb, isem=isb, gsem=gsemb, wsem=wsemb),
    ]

    wid = lax.axis_index("s") * NC + lax.axis_index("c")

    pltpu.sync_copy(wflat, wbuf)
    pltpu.sync_copy(bflat, bbuf)
    wv = [wbuf[pl.ds(h * 16, 16)] for h in range(2 * NG)]
    bv = [bbuf[pl.ds(h * 16, 16)] for h in range(2 * NG)]

    def coords(j):
        base = wid * RPW + j * CHUNK
        l = base // B
        b0 = base - l * B
        return base, l, b0

    def idx_copies(j, s):
        base, l, b0 = coords(j)
        lt = l // 8
        sub = l - lt * 8
        bt = b0 // 128
        cps = [pltpu.make_async_copy(cats[i].at[lt, bt, sub, :],
                                     s["idx"][i], s["isem"])
               for i in range(NCAT)]
        cps.append(pltpu.make_async_copy(
            xpad.at[pl.ds(base, CHUNK), :], s["xbuf"], s["isem"]))
        return cps

    def start(j, s):
        for cp in idx_copies(j, s):
            cp.start()

    def out_copies(j, s):
        base, l, b0 = coords(j)
        cps = [pltpu.make_async_copy(
            s["gbuf"][i],
            out.at[l, pl.ds(b0, CHUNK), pl.ds(i * D, D)], s["wsem"])
            for i in range(NCAT)]
        cps.append(pltpu.make_async_copy(
            s["ybuf"],
            out.at[l, pl.ds(b0, CHUNK), pl.ds(NCAT * D, CY)], s["wsem"]))
        return cps

    def finish(j, s):
        for cp in idx_copies(j, s):
            cp.wait()

        @pl.when(j >= 2)
        def _():
            for cp in out_copies(j, s):
                cp.wait()

        gcps = [pltpu.make_async_copy(tables[i].at[s["idx"][i]],
                                      s["gbuf"][i], s["gsem"])
                for i in range(NCAT)]
        for cp in gcps:
            cp.start()

        ybuf = s["ybuf"]

        def row_body(r, _c):
            xrow = s["xbuf"][r, :]
            for g in range(NG):
                xv = jnp.full((16,), xrow[g], jnp.float32)
                for h in range(2):
                    q = 2 * g + h
                    ybuf[r, pl.ds(q * 16, 16)] = xv * wv[q] + bv[q]
            return _c
        lax.fori_loop(0, CHUNK, row_body, 0)

        for cp in gcps:
            cp.wait()
        for cp in out_copies(j, s):
            cp.start()

    start(0, sets[0])
    start(1, sets[1])

    def iter_body(k, _c):
        finish(2 * k, sets[0])

        @pl.when(k < NCHUNK // 2 - 1)
        def _():
            start(2 * k + 2, sets[0])
        finish(2 * k + 1, sets[1])

        @pl.when(k < NCHUNK // 2 - 1)
        def _():
            start(2 * k + 3, sets[1])
        return _c

    lax.fori_loop(0, NCHUNK // 2, iter_body, 0)

    for cp in out_copies(NCHUNK - 2, sets[0]):
        cp.wait()
    for cp in out_copies(NCHUNK - 1, sets[1]):
        cp.wait()


@jax.jit
def _sc_call(cats_t, xpad, tables, wflat, bflat):
    mesh = plsc.VectorSubcoreMesh(core_axis_name="c", subcore_axis_name="s")
    scratch = (
        [pltpu.VMEM((CHUNK,), jnp.int32) for _ in range(2 * NCAT)]
        + [pltpu.VMEM((CHUNK, D), jnp.float32) for _ in range(2 * NCAT)]
        + [pltpu.VMEM((CHUNK, 16), jnp.float32) for _ in range(2)]
        + [pltpu.VMEM((CHUNK, CY), jnp.float32) for _ in range(2)]
        + [pltpu.VMEM((CY,), jnp.float32),
           pltpu.VMEM((CY,), jnp.float32)]
        + [pltpu.SemaphoreType.DMA for _ in range(6)]
    )
    fn = pl.kernel(
        _sc_body,
        out_type=jax.ShapeDtypeStruct((L, B, CTOT), jnp.float32),
        mesh=mesh,
        scratch_types=scratch,
        compiler_params=pltpu.CompilerParams(use_tc_tiling_on_sc=False,
                                             needs_layout_passes=False),
    )
    return fn(*cats_t, xpad, wflat, bflat, *tables)


def kernel(cat0, cat1, cat2, cat3, cat4, cat5, cat6, cat7,
           num_features, time, lengths,
           table0, table1, table2, table3, table4, table5, table6, table7,
           W, b):
    cats_t = [c.astype(jnp.int32).reshape(LT, 8, BT, 128).swapaxes(1, 2)
              for c in (cat0, cat1, cat2, cat3, cat4, cat5, cat6, cat7)]
    tables = [table0, table1, table2, table3, table4, table5, table6, table7]
    xpad = jnp.concatenate(
        [num_features.reshape(ROWS, NNUM),
         time.reshape(ROWS, 1).astype(jnp.float32),
         jnp.zeros((ROWS, 16 - NG), jnp.float32)], axis=1)
    return _sc_call(cats_t, xpad, tables, W.reshape(CY), b.reshape(CY))


# speedup vs baseline: 1.4256x; 1.0517x over previous
import functools

import jax
import jax.numpy as jnp
from jax import lax
from jax.experimental import pallas as pl
from jax.experimental.pallas import tpu as pltpu
from jax.experimental.pallas import tpu_sc as plsc

L = 200
B = 1024
V = 100000
NCAT = 8
D = 32
NNUM = 4
NED = 32
ROWS = L * B
NG = NNUM + 1
CY = NG * NED
CTOT = NCAT * D + CY
LT = L // 8
BT = B // 128

_info = plsc.get_sparse_core_info()
NC, NS = _info.num_cores, _info.num_subcores
NW = NC * NS
RPW = ROWS // NW
CHUNK = 128
NCHUNK = RPW // CHUNK


def _sc_body(c0, c1, c2, c3, c4, c5, c6, c7,
             xt, wflat, bflat,
             t0, t1, t2, t3, t4, t5, t6, t7,
             out,
             ia0, ia1, ia2, ia3, ia4, ia5, ia6, ia7,
             ib0, ib1, ib2, ib3, ib4, ib5, ib6, ib7,
             ga0, ga1, ga2, ga3, ga4, ga5, ga6, ga7,
             gb0, gb1, gb2, gb3, gb4, gb5, gb6, gb7,
             xta, xtb, xa, xb, ya, yb, wbuf, bbuf,
             isa, isb, gsema, gsemb, wsema, wsemb):
    cats = [c0, c1, c2, c3, c4, c5, c6, c7]
    tables = [t0, t1, t2, t3, t4, t5, t6, t7]
    sets = [
        dict(idx=[ia0, ia1, ia2, ia3, ia4, ia5, ia6, ia7],
             gbuf=[ga0, ga1, ga2, ga3, ga4, ga5, ga6, ga7],
             xtb=xta, xbuf=xa, ybuf=ya, isem=isa, gsem=gsema, wsem=wsema),
        dict(idx=[ib0, ib1, ib2, ib3, ib4, ib5, ib6, ib7],
             gbuf=[gb0, gb1, gb2, gb3, gb4, gb5, gb6, gb7],
             xtb=xtb, xbuf=xb, ybuf=yb, isem=isb, gsem=gsemb, wsem=wsemb),
    ]

    wid = lax.axis_index("s") * NC + lax.axis_index("c")

    pltpu.sync_copy(wflat, wbuf)
    pltpu.sync_copy(bflat, bbuf)
    wv = [wbuf[pl.ds(h * 16, 16)] for h in range(2 * NG)]
    bv = [bbuf[pl.ds(h * 16, 16)] for h in range(2 * NG)]
    iota = lax.iota(jnp.int32, 16)

    def coords(j):
        base = wid * RPW + j * CHUNK
        l = base // B
        b0 = base - l * B
        return base, l, b0

    def idx_copies(j, s):
        base, l, b0 = coords(j)
        lt = l // 8
        sub = l - lt * 8
        bt = b0 // 128
        cps = [pltpu.make_async_copy(cats[i].at[lt, bt, sub, :],
                                     s["idx"][i], s["isem"])
               for i in range(NCAT)]
        cps.append(pltpu.make_async_copy(
            xt.at[:, pl.ds(base, CHUNK)], s["xtb"], s["isem"]))
        return cps

    def start(j, s):
        for cp in idx_copies(j, s):
            cp.start()

    def out_copies(j, s):
        base, l, b0 = coords(j)
        cps = [pltpu.make_async_copy(
            s["gbuf"][i],
            out.at[l, pl.ds(b0, CHUNK), pl.ds(i * D, D)], s["wsem"])
            for i in range(NCAT)]
        cps.append(pltpu.make_async_copy(
            s["ybuf"],
            out.at[l, pl.ds(b0, CHUNK), pl.ds(NCAT * D, CY)], s["wsem"]))
        return cps

    def finish(j, s):
        for cp in idx_copies(j, s):
            cp.wait()

        @pl.when(j >= 2)
        def _():
            for cp in out_copies(j, s):
                cp.wait()

        gcps = [pltpu.make_async_copy(tables[i].at[s["idx"][i]],
                                      s["gbuf"][i], s["gsem"])
                for i in range(NCAT)]
        for cp in gcps:
            cp.start()

        for g in range(NG):
            colv = jnp.full((16,), g, jnp.int32)
            for m in range(8):
                v = s["xtb"][g, pl.ds(16 * m, 16)]
                plsc.store_scatter(s["xbuf"], [16 * m + iota, colv], v)

        ybuf = s["ybuf"]

        def row_body(r, _c):
            xrow = s["xbuf"][r, :]
            for g in range(NG):
                xv = jnp.full((16,), xrow[g], jnp.float32)
                for h in range(2):
                    q = 2 * g + h
                    ybuf[r, pl.ds(q * 16, 16)] = xv * wv[q] + bv[q]
            return _c
        lax.fori_loop(0, CHUNK, row_body, 0)

        for cp in gcps:
            cp.wait()
        for cp in out_copies(j, s):
            cp.start()

    start(0, sets[0])
    start(1, sets[1])

    def iter_body(k, _c):
        finish(2 * k, sets[0])

        @pl.when(k < NCHUNK // 2 - 1)
        def _():
            start(2 * k + 2, sets[0])
        finish(2 * k + 1, sets[1])

        @pl.when(k < NCHUNK // 2 - 1)
        def _():
            start(2 * k + 3, sets[1])
        return _c

    lax.fori_loop(0, NCHUNK // 2, iter_body, 0)

    for cp in out_copies(NCHUNK - 2, sets[0]):
        cp.wait()
    for cp in out_copies(NCHUNK - 1, sets[1]):
        cp.wait()


@jax.jit
def _sc_call(cats_t, xt, tables, wflat, bflat):
    mesh = plsc.VectorSubcoreMesh(core_axis_name="c", subcore_axis_name="s")
    scratch = (
        [pltpu.VMEM((CHUNK,), jnp.int32) for _ in range(2 * NCAT)]
        + [pltpu.VMEM((CHUNK, D), jnp.float32) for _ in range(2 * NCAT)]
        + [pltpu.VMEM((NG, CHUNK), jnp.float32) for _ in range(2)]
        + [pltpu.VMEM((CHUNK, 16), jnp.float32) for _ in range(2)]
        + [pltpu.VMEM((CHUNK, CY), jnp.float32) for _ in range(2)]
        + [pltpu.VMEM((CY,), jnp.float32),
           pltpu.VMEM((CY,), jnp.float32)]
        + [pltpu.SemaphoreType.DMA for _ in range(6)]
    )
    fn = pl.kernel(
        _sc_body,
        out_type=jax.ShapeDtypeStruct((L, B, CTOT), jnp.float32),
        mesh=mesh,
        scratch_types=scratch,
        compiler_params=pltpu.CompilerParams(use_tc_tiling_on_sc=False,
                                             needs_layout_passes=False),
    )
    return fn(*cats_t, xt, wflat, bflat, *tables)


def kernel(cat0, cat1, cat2, cat3, cat4, cat5, cat6, cat7,
           num_features, time, lengths,
           table0, table1, table2, table3, table4, table5, table6, table7,
           W, b):
    cats_t = [c.astype(jnp.int32).reshape(LT, 8, BT, 128).swapaxes(1, 2)
              for c in (cat0, cat1, cat2, cat3, cat4, cat5, cat6, cat7)]
    tables = [table0, table1, table2, table3, table4, table5, table6, table7]
    xt = jnp.concatenate(
        [num_features.reshape(ROWS, NNUM).T,
         time.reshape(1, ROWS).astype(jnp.float32)], axis=0)
    return _sc_call(cats_t, xt, tables, W.reshape(CY), b.reshape(CY))


# speedup vs baseline: 1.4275x; 1.0013x over previous
import jax
import jax.numpy as jnp
from jax import lax
from jax.experimental import pallas as pl
from jax.experimental.pallas import tpu as pltpu
from jax.experimental.pallas import tpu_sc as plsc

L = 200
B = 1024
V = 100000
NCAT = 8
D = 32
NNUM = 4
NED = 32
ROWS = L * B
NG = NNUM + 1
CY = NG * NED
CTOT = NCAT * D + CY
LT = L // 8
BT = B // 128

_info = plsc.get_sparse_core_info()
NC, NS = _info.num_cores, _info.num_subcores
NW = NC * NS
RPW = ROWS // NW
CHUNK = 128
NCHUNK = RPW // CHUNK


def _sc_body(c0, c1, c2, c3, c4, c5, c6, c7,
             xt, wflat, bflat,
             t0, t1, t2, t3, t4, t5, t6, t7,
             out,
             ia0, ia1, ia2, ia3, ia4, ia5, ia6, ia7,
             ib0, ib1, ib2, ib3, ib4, ib5, ib6, ib7,
             ga0, ga1, ga2, ga3, ga4, ga5, ga6, ga7,
             gb0, gb1, gb2, gb3, gb4, gb5, gb6, gb7,
             xta, xtb, xa, xb, ya, yb, wbuf, bbuf,
             isa, isb, gsema, gsemb, wsema, wsemb):
    cats = [c0, c1, c2, c3, c4, c5, c6, c7]
    tables = [t0, t1, t2, t3, t4, t5, t6, t7]
    sets = [
        dict(idx=[ia0, ia1, ia2, ia3, ia4, ia5, ia6, ia7],
             gbuf=[ga0, ga1, ga2, ga3, ga4, ga5, ga6, ga7],
             xtb=xta, xbuf=xa, ybuf=ya, isem=isa, gsem=gsema, wsem=wsema),
        dict(idx=[ib0, ib1, ib2, ib3, ib4, ib5, ib6, ib7],
             gbuf=[gb0, gb1, gb2, gb3, gb4, gb5, gb6, gb7],
             xtb=xtb, xbuf=xb, ybuf=yb, isem=isb, gsem=gsemb, wsem=wsemb),
    ]

    wid = lax.axis_index("s") * NC + lax.axis_index("c")

    pltpu.sync_copy(wflat, wbuf)
    pltpu.sync_copy(bflat, bbuf)
    wv = [wbuf[pl.ds(h * 16, 16)] for h in range(2 * NG)]
    bv = [bbuf[pl.ds(h * 16, 16)] for h in range(2 * NG)]
    iota = lax.iota(jnp.int32, 16)

    def coords(j):
        base = wid * RPW + j * CHUNK
        l = base // B
        b0 = base - l * B
        return base, l, b0

    def idx_copies(j, s):
        base, l, b0 = coords(j)
        lt = l // 8
        sub = l - lt * 8
        bt = b0 // 128
        cps = [pltpu.make_async_copy(cats[i].at[lt, bt, sub, :],
                                     s["idx"][i], s["isem"])
               for i in range(NCAT)]
        cps.append(pltpu.make_async_copy(
            xt.at[:, pl.ds(base, CHUNK)], s["xtb"], s["isem"]))
        return cps

    def start(j, s):
        for cp in idx_copies(j, s):
            cp.start()

    def out_copies(j, s):
        base, l, b0 = coords(j)
        cps = [pltpu.make_async_copy(
            s["gbuf"][i],
            out.at[l, pl.ds(b0, CHUNK), pl.ds(i * D, D)], s["wsem"])
            for i in range(NCAT)]
        cps.append(pltpu.make_async_copy(
            s["ybuf"],
            out.at[l, pl.ds(b0, CHUNK), pl.ds(NCAT * D, CY)], s["wsem"]))
        return cps

    def finish(j, s):
        for cp in idx_copies(j, s):
            cp.wait()

        @pl.when(j >= 2)
        def _():
            for cp in out_copies(j, s):
                cp.wait()

        gcps = [pltpu.make_async_copy(tables[i].at[s["idx"][i]],
                                      s["gbuf"][i], s["gsem"])
                for i in range(NCAT)]
        for cp in gcps:
            cp.start()

        for g in range(NG):
            colv = jnp.full((16,), g, jnp.int32)
            for m in range(8):
                v = s["xtb"][g, pl.ds(16 * m, 16)]
                plsc.store_scatter(s["xbuf"], [16 * m + iota, colv], v)

        ybuf = s["ybuf"]

        def row_body(r, _c):
            xrow = s["xbuf"][r, :]
            for g in range(NG):
                xv = jnp.full((16,), xrow[g], jnp.float32)
                for h in range(2):
                    q = 2 * g + h
                    ybuf[r, pl.ds(q * 16, 16)] = xv * wv[q] + bv[q]
            return _c
        lax.fori_loop(0, CHUNK, row_body, 0)

        for cp in gcps:
            cp.wait()
        for cp in out_copies(j, s):
            cp.start()

    start(0, sets[0])
    start(1, sets[1])

    def iter_body(k, _c):
        finish(2 * k, sets[0])

        @pl.when(k < NCHUNK // 2 - 1)
        def _():
            start(2 * k + 2, sets[0])
        finish(2 * k + 1, sets[1])

        @pl.when(k < NCHUNK // 2 - 1)
        def _():
            start(2 * k + 3, sets[1])
        return _c

    lax.fori_loop(0, NCHUNK // 2, iter_body, 0)

    for cp in out_copies(NCHUNK - 2, sets[0]):
        cp.wait()
    for cp in out_copies(NCHUNK - 1, sets[1]):
        cp.wait()


@jax.jit
def _sc_call(cats_t, xt, tables, wflat, bflat):
    mesh = plsc.VectorSubcoreMesh(core_axis_name="c", subcore_axis_name="s")
    scratch = (
        [pltpu.VMEM((CHUNK,), jnp.int32) for _ in range(2 * NCAT)]
        + [pltpu.VMEM((CHUNK, D), jnp.float32) for _ in range(2 * NCAT)]
        + [pltpu.VMEM((NG, CHUNK), jnp.float32) for _ in range(2)]
        + [pltpu.VMEM((CHUNK, 16), jnp.float32) for _ in range(2)]
        + [pltpu.VMEM((CHUNK, CY), jnp.float32) for _ in range(2)]
        + [pltpu.VMEM((CY,), jnp.float32),
           pltpu.VMEM((CY,), jnp.float32)]
        + [pltpu.SemaphoreType.DMA for _ in range(6)]
    )
    fn = pl.kernel(
        _sc_body,
        out_type=jax.ShapeDtypeStruct((L, B, CTOT), jnp.float32),
        mesh=mesh,
        scratch_types=scratch,
        compiler_params=pltpu.CompilerParams(use_tc_tiling_on_sc=False,
                                             needs_layout_passes=False),
    )
    return fn(*cats_t, xt, wflat, bflat, *tables)


def kernel(cat0, cat1, cat2, cat3, cat4, cat5, cat6, cat7,
           num_features, time, lengths,
           table0, table1, table2, table3, table4, table5, table6, table7,
           W, b):
    cats_t = [c.astype(jnp.int32).reshape(LT, 8, BT, 128).swapaxes(1, 2)
              for c in (cat0, cat1, cat2, cat3, cat4, cat5, cat6, cat7)]
    tables = [table0, table1, table2, table3, table4, table5, table6, table7]
    xt = jnp.concatenate(
        [num_features.reshape(ROWS, NNUM).T,
         time.reshape(1, ROWS).astype(jnp.float32)], axis=0)
    return _sc_call(cats_t, xt, tables, W.reshape(CY), b.reshape(CY))
